# baseline (speedup 1.0000x reference)
import functools

import jax
import jax.numpy as jnp
from jax import lax
from jax.experimental import pallas as pl
from jax.experimental.pallas import tpu as pltpu
from jax.experimental.pallas import tpu_sc as plsc

_E = 320000
_T = 8
_D = 128
_EPS = 1e-5

_NC = 2
_NS = 16
_NW = _NC * _NS
_BPW = _E // _NW
_C = 80
_NCHUNK = _BPW // _C
_NBUF = 5
_OUTER = _NCHUNK // _NBUF
_L = 16


def _prep_body(table_ref, gamma_ref, beta_ref, out_ref):
    emb = table_ref[...] * (_D ** 0.5)
    mean = jnp.mean(emb, axis=-1, keepdims=True)
    cen = emb - mean
    var = jnp.mean(cen * cen, axis=-1, keepdims=True)
    out_ref[...] = cen * lax.rsqrt(var + _EPS) * gamma_ref[...] + beta_ref[...]


def _prep(table, gamma, beta):
    return pl.pallas_call(
        _prep_body,
        out_shape=jax.ShapeDtypeStruct((_T, _D), jnp.float32),
    )(table, gamma, beta)


_mesh = plsc.VectorSubcoreMesh(core_axis_name="c", subcore_axis_name="s")


@functools.partial(
    pl.kernel,
    mesh=_mesh,
    out_type=jax.ShapeDtypeStruct((_E, _D), jnp.float32),
    compiler_params=pltpu.CompilerParams(needs_layout_passes=False),
    scratch_types=[
        pltpu.VMEM((_T, _D), jnp.float32),
        pltpu.VMEM_SHARED((_T, _D), jnp.float32),
        pltpu.VMEM((_BPW,), jnp.int32),
        pltpu.VMEM((_NBUF * _C, _D), jnp.float32),
        pltpu.SemaphoreType.DMA,
        pltpu.SemaphoreType.DMA((_NBUF,)),
        pltpu.SemaphoreType.DMA((_NBUF,)),
    ],
)
def _expand(ids_hbm, p_hbm, out_hbm, p_v, p_sh, idx_v, rows_v,
            isem, gsem, ssem):
    cid = lax.axis_index("c")
    sid = lax.axis_index("s")
    wid = sid * _NC + cid
    base = wid * _BPW

    ids_cp = pltpu.make_async_copy(
        ids_hbm.at[pl.ds(base, _BPW)], idx_v, isem)
    ids_cp.start()

    @pl.when(sid == 0)
    def _pub():
        pltpu.sync_copy(p_hbm, p_v)
        pltpu.sync_copy(p_v, p_sh)

    plsc.subcore_barrier()
    ids_cp.wait()

    def gather_copy(j, b):
        off = pl.multiple_of(j * _C, 8)
        return pltpu.make_async_copy(
            p_sh.at[idx_v.at[pl.ds(off, _C)]],
            rows_v.at[pl.ds(b * _C, _C)],
            gsem.at[b])

    def store_copy(j, b):
        off = pl.multiple_of(base + j * _C, 8)
        return pltpu.make_async_copy(
            rows_v.at[pl.ds(b * _C, _C)],
            out_hbm.at[pl.ds(off, _C)],
            ssem.at[b])

    for b in range(_NBUF):
        gather_copy(b, b).start()
    for b in range(_NBUF):
        gather_copy(b, b).wait()
        store_copy(b, b).start()

    def outer(grp, carry):
        jn = grp * _NBUF
        for b in range(_NBUF):
            store_copy(jn - _NBUF + b, b).wait()
            gather_copy(jn + b, b).start()
        for b in range(_NBUF):
            gather_copy(jn + b, b).wait()
            store_copy(jn + b, b).start()
        return carry

    lax.fori_loop(1, _OUTER, outer, 0)

    jlast = (_OUTER - 1) * _NBUF
    for b in range(_NBUF):
        store_copy(jlast + b, b).wait()


def kernel(edge_type_ids, table, gamma, beta):
    p = _prep(table.astype(jnp.float32), gamma.astype(jnp.float32),
              beta.astype(jnp.float32))
    return _expand(edge_type_ids.astype(jnp.int32), p)


# speedup vs baseline: 1.0066x; 1.0066x over previous
import functools

import jax
import jax.numpy as jnp
from jax import lax
from jax.experimental import pallas as pl
from jax.experimental.pallas import tpu as pltpu
from jax.experimental.pallas import tpu_sc as plsc

_E = 320000
_T = 8
_D = 128
_EPS = 1e-5

_NC = 2
_NS = 16
_NW = _NC * _NS
_BPW = _E // _NW
_C = 40
_NCHUNK = _BPW // _C
_NBUF = 10
_OUTER = _NCHUNK // _NBUF
_L = 16


def _prep_body(table_ref, gamma_ref, beta_ref, out_ref):
    emb = table_ref[...] * (_D ** 0.5)
    mean = jnp.mean(emb, axis=-1, keepdims=True)
    cen = emb - mean
    var = jnp.mean(cen * cen, axis=-1, keepdims=True)
    out_ref[...] = cen * lax.rsqrt(var + _EPS) * gamma_ref[...] + beta_ref[...]


def _prep(table, gamma, beta):
    return pl.pallas_call(
        _prep_body,
        out_shape=jax.ShapeDtypeStruct((_T, _D), jnp.float32),
    )(table, gamma, beta)


_mesh = plsc.VectorSubcoreMesh(core_axis_name="c", subcore_axis_name="s")


@functools.partial(
    pl.kernel,
    mesh=_mesh,
    out_type=jax.ShapeDtypeStruct((_E, _D), jnp.float32),
    compiler_params=pltpu.CompilerParams(needs_layout_passes=False),
    scratch_types=[
        pltpu.VMEM((_T, _D), jnp.float32),
        pltpu.VMEM_SHARED((_T, _D), jnp.float32),
        pltpu.VMEM((_BPW,), jnp.int32),
        pltpu.VMEM((_NBUF * _C, _D), jnp.float32),
        pltpu.SemaphoreType.DMA,
        pltpu.SemaphoreType.DMA((_NBUF,)),
        pltpu.SemaphoreType.DMA((_NBUF,)),
    ],
)
def _expand(ids_hbm, p_hbm, out_hbm, p_v, p_sh, idx_v, rows_v,
            isem, gsem, ssem):
    cid = lax.axis_index("c")
    sid = lax.axis_index("s")
    wid = sid * _NC + cid
    base = wid * _BPW

    ids_cp = pltpu.make_async_copy(
        ids_hbm.at[pl.ds(base, _BPW)], idx_v, isem)
    ids_cp.start()

    @pl.when(sid == 0)
    def _pub():
        pltpu.sync_copy(p_hbm, p_v)
        pltpu.sync_copy(p_v, p_sh)

    plsc.subcore_barrier()
    ids_cp.wait()

    def gather_copy(j, b):
        off = pl.multiple_of(j * _C, 8)
        return pltpu.make_async_copy(
            p_sh.at[idx_v.at[pl.ds(off, _C)]],
            rows_v.at[pl.ds(b * _C, _C)],
            gsem.at[b])

    def store_copy(j, b):
        off = pl.multiple_of(base + j * _C, 8)
        return pltpu.make_async_copy(
            rows_v.at[pl.ds(b * _C, _C)],
            out_hbm.at[pl.ds(off, _C)],
            ssem.at[b])

    for b in range(_NBUF):
        gather_copy(b, b).start()
    for b in range(_NBUF):
        gather_copy(b, b).wait()
        store_copy(b, b).start()

    def outer(grp, carry):
        jn = grp * _NBUF
        for b in range(_NBUF):
            store_copy(jn - _NBUF + b, b).wait()
            gather_copy(jn + b, b).start()
        for b in range(_NBUF):
            gather_copy(jn + b, b).wait()
            store_copy(jn + b, b).start()
        return carry

    lax.fori_loop(1, _OUTER, outer, 0)

    jlast = (_OUTER - 1) * _NBUF
    for b in range(_NBUF):
        store_copy(jlast + b, b).wait()


def kernel(edge_type_ids, table, gamma, beta):
    p = _prep(table.astype(jnp.float32), gamma.astype(jnp.float32),
              beta.astype(jnp.float32))
    return _expand(edge_type_ids.astype(jnp.int32), p)
